# baseline (speedup 1.0000x reference)
import functools

import jax
import jax.numpy as jnp
from jax import lax
from jax.experimental import pallas as pl
from jax.experimental.pallas import tpu as pltpu
from jax.experimental.pallas import tpu_sc as plsc

B = 16384
D = 32
NC = 2
NS = 16
NW = NC * NS
BPW = B // NW
NCHUNK = 8
HALF = BPW // NCHUNK
NSLOT = 4
NG = BPW // 16
GPC = HALF // 16
NMAX = 1000000


def _sc_gather2_body(uidx_hbm, iidx_hbm, t_u, t_i,
                     o_u, o_i,
                     uidx_v, iidx_v, blks, su, si, sems):
    wid = lax.axis_index("s") * NC + lax.axis_index("c")
    base = wid * BPW
    pltpu.sync_copy(uidx_hbm.at[pl.ds(base, BPW)], uidx_v.at[pl.ds(0, BPW)])
    pltpu.sync_copy(iidx_hbm.at[pl.ds(base, BPW)], iidx_v.at[pl.ds(0, BPW)])

    rows0 = lax.iota(jnp.int32, 16)
    rows1 = rows0 + 16

    def load_group(gk):
        uvec = jnp.clip(uidx_v[pl.ds(gk, 16)], 0, NMAX)
        ivec = jnp.clip(iidx_v[pl.ds(gk, 16)], 0, NMAX)
        cbu = (uvec // 128) * 128
        cbi = (ivec // 128) * 128
        return cbu, cbi, uvec - cbu, ivec - cbi

    def fire(cbu, cbi, s, slot):
        cu = pl.multiple_of(cbu[s], 128)
        ci = pl.multiple_of(cbi[s], 128)
        sem = sems.at[slot]
        pltpu.async_copy(t_u.at[:, pl.ds(cu, 128)], blks.at[slot, 0], sem)
        pltpu.async_copy(t_i.at[:, pl.ds(ci, 128)], blks.at[slot, 1], sem)

    def wait_slot(slot):
        for t in range(2):
            pltpu.make_async_copy(
                t_u.at[:, pl.ds(0, 128)], blks.at[slot, t], sems.at[slot]
            ).wait()

    def extract(colu, coli, s, slot, r):
        cu = jnp.broadcast_to(colu[s], (16,))
        ci = jnp.broadcast_to(coli[s], (16,))
        for t, (stage, cv) in enumerate(((su, cu), (si, ci))):
            v0 = plsc.load_gather(blks.at[slot, t], [rows0, cv])
            v1 = plsc.load_gather(blks.at[slot, t], [rows1, cv])
            stage[r, 0:16] = v0
            stage[r, 16:32] = v1

    first = load_group(0)
    for s in range(NSLOT):
        fire(first[0], first[1], s, s)

    def group(g, carry):
        cbu, cbi, colu, coli = carry
        nxt = load_group((g + 1) * 16)
        for s in range(16):
            slot = s % NSLOT
            wait_slot(slot)
            extract(colu, coli, s, slot, (g % GPC) * 16 + s)
            if s + NSLOT < 16:
                fire(cbu, cbi, s + NSLOT, slot)
            else:

                @pl.when(g + 1 < NG)
                def _prefire(s=s, slot=slot, nxt=nxt):
                    fire(nxt[0], nxt[1], s + NSLOT - 16, slot)

        @pl.when(g % GPC == GPC - 1)
        def _flush(g=g):
            rows = pl.ds(base + (g // GPC) * HALF, HALF)
            pltpu.sync_copy(su, o_u.at[rows, :])
            pltpu.sync_copy(si, o_i.at[rows, :])

        return nxt

    lax.fori_loop(0, NG, group, first)


@functools.cache
def _sc_gather2():
    return pl.kernel(
        _sc_gather2_body,
        out_type=[jax.ShapeDtypeStruct((B, D), jnp.float32) for _ in range(2)],
        mesh=plsc.VectorSubcoreMesh(core_axis_name="c", subcore_axis_name="s"),
        scratch_types=[
            pltpu.VMEM((BPW + 16,), jnp.int32),
            pltpu.VMEM((BPW + 16,), jnp.int32),
            pltpu.VMEM((NSLOT, 2, D, 128), jnp.float32),
            pltpu.VMEM((HALF, D), jnp.float32),
            pltpu.VMEM((HALF, D), jnp.float32),
            pltpu.SemaphoreType.DMA((NSLOT,)),
        ],
        compiler_params=pltpu.CompilerParams(needs_layout_passes=False),
    )


def _gelu(x):
    return 0.5 * x * (1.0 + lax.erf(x * (2.0 ** -0.5)))


def _bn_gelu(z, g, be):
    m = jnp.mean(z, axis=0, keepdims=True)
    v = jnp.mean((z - m) ** 2, axis=0, keepdims=True)
    return _gelu(g * (z - m) * lax.rsqrt(v + 1e-5) + be)


def _tower_body(ue_ref, ie_ref,
                W1_ref, b1_ref, g1_ref, be1_ref,
                W2_ref, b2_ref, g2_ref, be2_ref,
                W3_ref, b3_ref, g3_ref, be3_ref,
                W4_ref, b4_ref, g4_ref, be4_ref, h_ref):
    W1 = W1_ref[...]
    z = (jnp.dot(ue_ref[...], W1[0:D, :], preferred_element_type=jnp.float32)
         + jnp.dot(ie_ref[...], W1[D:2 * D, :], preferred_element_type=jnp.float32)
         + b1_ref[...])
    h = _bn_gelu(z, g1_ref[...], be1_ref[...])
    z = jnp.dot(h, W2_ref[...], preferred_element_type=jnp.float32) + b2_ref[...]
    h = _bn_gelu(z, g2_ref[...], be2_ref[...])
    z = jnp.dot(h, W3_ref[...], preferred_element_type=jnp.float32) + b3_ref[...]
    h = _bn_gelu(z, g3_ref[...], be3_ref[...])
    z = jnp.dot(h, W4_ref[...], preferred_element_type=jnp.float32) + b4_ref[...]
    h_ref[...] = _bn_gelu(z, g4_ref[...], be4_ref[...])


def _head_body(h_ref, uf_ref, if_ref, woh_ref, wop_ref, bo_ref, out_ref):
    prod = uf_ref[...] * if_ref[...]
    logits = (jnp.sum(h_ref[...] * woh_ref[...], axis=1, keepdims=True)
              + jnp.sum(prod * wop_ref[...], axis=1, keepdims=True)
              + bo_ref[...])
    out_ref[...] = jax.nn.sigmoid(logits)


def kernel(user_indices, item_indices, emb_user_mlp, emb_item_mlp, emb_user_mf,
           emb_item_mf, W1, b1, g1, be1, W2, b2, g2, be2, W3, b3, g3, be3,
           W4, b4, g4, be4, Wo, bo):
    ui = user_indices.astype(jnp.int32)
    ii = item_indices.astype(jnp.int32)
    gather2 = _sc_gather2()
    ue, ie = gather2(ui, ii, emb_user_mlp.T, emb_item_mlp.T)
    uf, if_ = gather2(ui, ii, emb_user_mf.T, emb_item_mf.T)
    h = pl.pallas_call(
        _tower_body,
        out_shape=jax.ShapeDtypeStruct((B, 8), jnp.float32),
        compiler_params=pltpu.CompilerParams(vmem_limit_bytes=100 * 1024 * 1024),
    )(ue, ie,
      W1, b1.reshape(1, -1), g1.reshape(1, -1), be1.reshape(1, -1),
      W2, b2.reshape(1, -1), g2.reshape(1, -1), be2.reshape(1, -1),
      W3, b3.reshape(1, -1), g3.reshape(1, -1), be3.reshape(1, -1),
      W4, b4.reshape(1, -1), g4.reshape(1, -1), be4.reshape(1, -1))
    out = pl.pallas_call(
        _head_body,
        out_shape=jax.ShapeDtypeStruct((B, 1), jnp.float32),
        compiler_params=pltpu.CompilerParams(vmem_limit_bytes=100 * 1024 * 1024),
    )(h, uf, if_, Wo[0:8, :].reshape(1, 8), Wo[8:40, :].reshape(1, D),
      bo.reshape(1, 1))
    return out


# speedup vs baseline: 1.0486x; 1.0486x over previous
import functools

import jax
import jax.numpy as jnp
from jax import lax
from jax.experimental import pallas as pl
from jax.experimental.pallas import tpu as pltpu
from jax.experimental.pallas import tpu_sc as plsc

B = 16384
D = 32
NC = 2
NS = 16
NW = NC * NS
BPW = B // NW
NCHUNK = 8
HALF = BPW // NCHUNK
NSLOT = 4
NG = BPW // 16
GPC = HALF // 16
NMAX = 1000000


def _sc_gather_body(uidx_hbm, iidx_hbm, t_umlp, t_imlp, t_umf, t_imf,
                    o_umlp, o_imlp, o_umf, o_imf,
                    uidx_v, iidx_v, blks, su1, si1, su2, si2, sems):
    wid = lax.axis_index("s") * NC + lax.axis_index("c")
    base = wid * BPW
    pltpu.sync_copy(uidx_hbm.at[pl.ds(base, BPW)], uidx_v.at[pl.ds(0, BPW)])
    pltpu.sync_copy(iidx_hbm.at[pl.ds(base, BPW)], iidx_v.at[pl.ds(0, BPW)])

    rows0 = lax.iota(jnp.int32, 16)
    rows1 = rows0 + 16

    def load_group(gk):
        uvec = jnp.clip(uidx_v[pl.ds(gk, 16)], 0, NMAX)
        ivec = jnp.clip(iidx_v[pl.ds(gk, 16)], 0, NMAX)
        cbu = (uvec // 128) * 128
        cbi = (ivec // 128) * 128
        return cbu, cbi, uvec - cbu, ivec - cbi

    def fire(cbu, cbi, s, slot):
        cu = pl.multiple_of(cbu[s], 128)
        ci = pl.multiple_of(cbi[s], 128)
        sem = sems.at[slot]
        pltpu.async_copy(t_umlp.at[:, pl.ds(cu, 128)], blks.at[slot, 0], sem)
        pltpu.async_copy(t_imlp.at[:, pl.ds(ci, 128)], blks.at[slot, 1], sem)
        pltpu.async_copy(t_umf.at[:, pl.ds(cu, 128)], blks.at[slot, 2], sem)
        pltpu.async_copy(t_imf.at[:, pl.ds(ci, 128)], blks.at[slot, 3], sem)

    def wait_slot(slot):
        for t in range(4):
            pltpu.make_async_copy(
                t_umlp.at[:, pl.ds(0, 128)], blks.at[slot, t], sems.at[slot]
            ).wait()

    def extract(colu, coli, s, slot, r):
        cu = jnp.broadcast_to(colu[s], (16,))
        ci = jnp.broadcast_to(coli[s], (16,))
        for t, (stage, cv) in enumerate(((su1, cu), (si1, ci), (su2, cu), (si2, ci))):
            v0 = plsc.load_gather(blks.at[slot, t], [rows0, cv])
            v1 = plsc.load_gather(blks.at[slot, t], [rows1, cv])
            stage[r, 0:16] = v0
            stage[r, 16:32] = v1

    first = load_group(0)
    for s in range(NSLOT):
        fire(first[0], first[1], s, s)

    def group(g, carry):
        cbu, cbi, colu, coli = carry
        nxt = load_group((g + 1) * 16)
        for s in range(16):
            slot = s % NSLOT
            wait_slot(slot)
            extract(colu, coli, s, slot, (g % GPC) * 16 + s)
            if s + NSLOT < 16:
                fire(cbu, cbi, s + NSLOT, slot)
            else:

                @pl.when(g + 1 < NG)
                def _prefire(s=s, slot=slot, nxt=nxt):
                    fire(nxt[0], nxt[1], s + NSLOT - 16, slot)

        @pl.when(g % GPC == GPC - 1)
        def _flush(g=g):
            rows = pl.ds(base + (g // GPC) * HALF, HALF)
            pltpu.sync_copy(su1, o_umlp.at[rows, :])
            pltpu.sync_copy(si1, o_imlp.at[rows, :])
            pltpu.sync_copy(su2, o_umf.at[rows, :])
            pltpu.sync_copy(si2, o_imf.at[rows, :])

        return nxt

    lax.fori_loop(0, NG, group, first)


@functools.cache
def _sc_gather():
    return pl.kernel(
        _sc_gather_body,
        out_type=[jax.ShapeDtypeStruct((B, D), jnp.float32) for _ in range(4)],
        mesh=plsc.VectorSubcoreMesh(core_axis_name="c", subcore_axis_name="s"),
        scratch_types=[
            pltpu.VMEM((BPW + 16,), jnp.int32),
            pltpu.VMEM((BPW + 16,), jnp.int32),
            pltpu.VMEM((NSLOT, 4, D, 128), jnp.float32),
            pltpu.VMEM((HALF, D), jnp.float32),
            pltpu.VMEM((HALF, D), jnp.float32),
            pltpu.VMEM((HALF, D), jnp.float32),
            pltpu.VMEM((HALF, D), jnp.float32),
            pltpu.SemaphoreType.DMA((NSLOT,)),
        ],
        compiler_params=pltpu.CompilerParams(needs_layout_passes=False),
    )


def _gelu(x):
    return 0.5 * x * (1.0 + lax.erf(x * (2.0 ** -0.5)))


def _bn_gelu(z, g, be):
    m = jnp.mean(z, axis=0, keepdims=True)
    v = jnp.mean((z - m) ** 2, axis=0, keepdims=True)
    return _gelu(g * (z - m) * lax.rsqrt(v + 1e-5) + be)


def _mlp_body(ue_ref, ie_ref, uf_ref, if_ref,
              W1_ref, b1_ref, g1_ref, be1_ref,
              W2_ref, b2_ref, g2_ref, be2_ref,
              W3_ref, b3_ref, g3_ref, be3_ref,
              W4_ref, b4_ref, g4_ref, be4_ref,
              woh_ref, wop_ref, bo_ref, out_ref):
    W1 = W1_ref[...]
    z = (jnp.dot(ue_ref[...], W1[0:D, :], preferred_element_type=jnp.float32)
         + jnp.dot(ie_ref[...], W1[D:2 * D, :], preferred_element_type=jnp.float32)
         + b1_ref[...])
    h = _bn_gelu(z, g1_ref[...], be1_ref[...])
    z = jnp.dot(h, W2_ref[...], preferred_element_type=jnp.float32) + b2_ref[...]
    h = _bn_gelu(z, g2_ref[...], be2_ref[...])
    z = jnp.dot(h, W3_ref[...], preferred_element_type=jnp.float32) + b3_ref[...]
    h = _bn_gelu(z, g3_ref[...], be3_ref[...])
    z = jnp.dot(h, W4_ref[...], preferred_element_type=jnp.float32) + b4_ref[...]
    h = _bn_gelu(z, g4_ref[...], be4_ref[...])
    prod = uf_ref[...] * if_ref[...]
    logits = (jnp.sum(h * woh_ref[...], axis=1, keepdims=True)
              + jnp.sum(prod * wop_ref[...], axis=1, keepdims=True)
              + bo_ref[...])
    out_ref[...] = jax.nn.sigmoid(logits)


def kernel(user_indices, item_indices, emb_user_mlp, emb_item_mlp, emb_user_mf,
           emb_item_mf, W1, b1, g1, be1, W2, b2, g2, be2, W3, b3, g3, be3,
           W4, b4, g4, be4, Wo, bo):
    ui = user_indices.astype(jnp.int32)
    ii = item_indices.astype(jnp.int32)
    ue, ie, uf, if_ = _sc_gather()(
        ui, ii, emb_user_mlp.T, emb_item_mlp.T, emb_user_mf.T, emb_item_mf.T)
    out = pl.pallas_call(
        _mlp_body,
        out_shape=jax.ShapeDtypeStruct((B, 1), jnp.float32),
        compiler_params=pltpu.CompilerParams(vmem_limit_bytes=100 * 1024 * 1024),
    )(ue, ie, uf, if_,
      W1, b1.reshape(1, -1), g1.reshape(1, -1), be1.reshape(1, -1),
      W2, b2.reshape(1, -1), g2.reshape(1, -1), be2.reshape(1, -1),
      W3, b3.reshape(1, -1), g3.reshape(1, -1), be3.reshape(1, -1),
      W4, b4.reshape(1, -1), g4.reshape(1, -1), be4.reshape(1, -1),
      Wo[0:8, :].reshape(1, 8), Wo[8:40, :].reshape(1, D), bo.reshape(1, 1))
    return out
